# baseline (speedup 1.0000x reference)
import functools

import jax
import jax.numpy as jnp
from jax import lax
from jax.experimental import pallas as pl
from jax.experimental.pallas import tpu as pltpu
from jax.experimental.pallas import tpu_sc as plsc

NN = 10000
NP = 10240
EE = 320000
CH = 96
NCHUNK = 108
EPT = CH * NCHUNK
NWORK = 32
EP = EPT * NWORK
DUMP = 10200
BLK = 1024


def _zero_tile_slice(grows0, w0, num_sh, den_sh, sid, zero_den):
    zeros16 = jnp.zeros((16,), jnp.float32)

    def zrow(r, c):
        for cc in range(8):
            grows0[r, pl.ds(cc * 16, 16)] = zeros16
        return c

    lax.fori_loop(0, CH, zrow, 0)
    for cc in range(6):
        w0[pl.ds(cc * 16, 16)] = zeros16

    r0 = sid * 640
    for k in range(10):
        pltpu.sync_copy(grows0.at[pl.ds(0, 64)],
                        num_sh.at[pl.ds(r0 + k * 64, 64)])
        if zero_den:
            pltpu.sync_copy(w0.at[pl.ds(0, 64)],
                            den_sh.at[pl.ds(r0 + k * 64, 64)])


def _edge_pass(src_hbm, dst_hbm, asrc_v, adst_v, h_hbm, n_out, den_out,
               p0, p1, num_sh, den_sh, cid, sid, do_den):
    ebase = (cid * 16 + sid) * EPT

    def drain(p):
        _, didx, w, grows, _, ssem = p
        pltpu.make_async_copy(grows, num_sh.at[didx], ssem).wait()
        if do_den:
            pltpu.make_async_copy(w, den_sh.at[didx], ssem).wait()

    def prep(j, p, dr):
        sidx, didx, w, grows, gsem, ssem = p
        if dr:
            drain(p)
        eb = ebase + j * CH
        pltpu.sync_copy(src_hbm.at[pl.ds(eb, CH)], sidx)
        pltpu.sync_copy(dst_hbm.at[pl.ds(eb, CH)], didx)
        pltpu.async_copy(h_hbm.at[sidx], grows, gsem)
        for t in range(CH // 16):
            si = sidx[pl.ds(t * 16, 16)]
            di = didx[pl.ds(t * 16, 16)]
            e = plsc.load_gather(asrc_v, [si]) + plsc.load_gather(adst_v, [di])
            e = jnp.maximum(e, 0.0) + 0.2 * jnp.minimum(e, 0.0)
            w[pl.ds(t * 16, 16)] = jnp.exp(e)

    def exc(p):
        sidx, didx, w, grows, gsem, ssem = p
        pltpu.make_async_copy(h_hbm.at[sidx], grows, gsem).wait()

        def scale(r, carry2):
            rr = 2 * r
            wa = plsc.load_gather(w, [jnp.zeros((16,), jnp.int32) + rr])
            wb = plsc.load_gather(w, [jnp.zeros((16,), jnp.int32) + (rr + 1)])
            for cc in range(8):
                grows[rr, pl.ds(cc * 16, 16)] = grows[rr, pl.ds(cc * 16, 16)] * wa
            for cc in range(8):
                grows[rr + 1, pl.ds(cc * 16, 16)] = grows[rr + 1, pl.ds(cc * 16, 16)] * wb
            return carry2

        lax.fori_loop(0, CH // 2, scale, 0)
        pltpu.async_copy(grows, num_sh.at[didx], ssem, add=True)
        if do_den:
            pltpu.async_copy(w, den_sh.at[didx], ssem, add=True)

    prep(0, p0, dr=False)
    prep(1, p1, dr=False)
    exc(p0)
    exc(p1)
    prep(2, p0, dr=True)
    prep(3, p1, dr=True)

    def super_iter(k, carry):
        exc(p0)
        exc(p1)
        prep(2 * k + 2, p0, dr=True)
        prep(2 * k + 3, p1, dr=True)
        return carry

    lax.fori_loop(1, NCHUNK // 2 - 1, super_iter, 0)
    exc(p0)
    exc(p1)
    drain(p0)
    drain(p1)
    plsc.subcore_barrier()

    r0 = sid * 640
    for k in range(5):
        pltpu.sync_copy(num_sh.at[pl.ds(r0 + k * 128, 128)],
                        n_out.at[cid, pl.ds(r0 + k * 128, 128)])
        if do_den:
            pltpu.sync_copy(den_sh.at[pl.ds(r0 + k * 128, 128)],
                            den_out.at[cid, pl.ds(r0 + k * 128, 128)])


def _sc_conv1_body(src_hbm, dst_hbm, asrc_hbm, adst_hbm, ha_hbm, hb_hbm,
                   numa_out, numb_out, den_out,
                   asrc_v, adst_v, sidx0, sidx1, didx0, didx1,
                   w0, w1, grows0, grows1,
                   gsem0, gsem1, ssem0, ssem1,
                   num_sh, den_sh):
    cid = lax.axis_index("c")
    sid = lax.axis_index("s")
    p0 = (sidx0, didx0, w0, grows0, gsem0, ssem0)
    p1 = (sidx1, didx1, w1, grows1, gsem1, ssem1)

    _zero_tile_slice(grows0, w0, num_sh, den_sh, sid, zero_den=True)
    pltpu.sync_copy(asrc_hbm, asrc_v)
    pltpu.sync_copy(adst_hbm, adst_v)
    plsc.subcore_barrier()

    _edge_pass(src_hbm, dst_hbm, asrc_v, adst_v, ha_hbm, numa_out,
               den_out, p0, p1, num_sh, den_sh, cid, sid, do_den=True)
    _zero_tile_slice(grows0, w0, num_sh, den_sh, sid, zero_den=False)
    plsc.subcore_barrier()
    _edge_pass(src_hbm, dst_hbm, asrc_v, adst_v, hb_hbm, numb_out,
               den_out, p0, p1, num_sh, den_sh, cid, sid, do_den=False)


def _sc_conv2_body(src_hbm, dst_hbm, asrc_hbm, adst_hbm, h_hbm,
                   num_out, den_out,
                   asrc_v, adst_v, sidx0, sidx1, didx0, didx1,
                   w0, w1, grows0, grows1,
                   gsem0, gsem1, ssem0, ssem1,
                   num_sh, den_sh):
    cid = lax.axis_index("c")
    sid = lax.axis_index("s")
    p0 = (sidx0, didx0, w0, grows0, gsem0, ssem0)
    p1 = (sidx1, didx1, w1, grows1, gsem1, ssem1)

    _zero_tile_slice(grows0, w0, num_sh, den_sh, sid, zero_den=True)
    pltpu.sync_copy(asrc_hbm, asrc_v)
    pltpu.sync_copy(adst_hbm, adst_v)
    plsc.subcore_barrier()
    _edge_pass(src_hbm, dst_hbm, asrc_v, adst_v, h_hbm, num_out,
               den_out, p0, p1, num_sh, den_sh, cid, sid, do_den=True)


def _sc_scratch():
    return [
            pltpu.VMEM((NP,), jnp.float32),
            pltpu.VMEM((NP,), jnp.float32),
            pltpu.VMEM((CH,), jnp.int32),
            pltpu.VMEM((CH,), jnp.int32),
            pltpu.VMEM((CH,), jnp.int32),
            pltpu.VMEM((CH,), jnp.int32),
            pltpu.VMEM((CH,), jnp.float32),
            pltpu.VMEM((CH,), jnp.float32),
            pltpu.VMEM((CH, 128), jnp.float32),
            pltpu.VMEM((CH, 128), jnp.float32),
            pltpu.SemaphoreType.DMA,
            pltpu.SemaphoreType.DMA,
            pltpu.SemaphoreType.DMA,
            pltpu.SemaphoreType.DMA,
            pltpu.VMEM_SHARED((NP, 128), jnp.float32),
            pltpu.VMEM_SHARED((NP,), jnp.float32),
    ]


@jax.jit
def _sc_conv1(src, dst, a_s, a_d, ha, hb):
    mesh = plsc.VectorSubcoreMesh(core_axis_name="c", subcore_axis_name="s")
    f = pl.kernel(
        _sc_conv1_body,
        mesh=mesh,
        compiler_params=pltpu.CompilerParams(needs_layout_passes=False),
        out_type=[
            jax.ShapeDtypeStruct((2, NP, 128), jnp.float32),
            jax.ShapeDtypeStruct((2, NP, 128), jnp.float32),
            jax.ShapeDtypeStruct((2, NP), jnp.float32),
        ],
        scratch_types=_sc_scratch(),
    )
    return f(src, dst, a_s, a_d, ha, hb)


@jax.jit
def _sc_conv2(src, dst, a_s, a_d, h):
    mesh = plsc.VectorSubcoreMesh(core_axis_name="c", subcore_axis_name="s")
    f = pl.kernel(
        _sc_conv2_body,
        mesh=mesh,
        compiler_params=pltpu.CompilerParams(needs_layout_passes=False),
        out_type=[
            jax.ShapeDtypeStruct((2, NP, 128), jnp.float32),
            jax.ShapeDtypeStruct((2, NP), jnp.float32),
        ],
        scratch_types=_sc_scratch(),
    )
    return f(src, dst, a_s, a_d, h)


def _dense1_body(x_ref, w1a_ref, w1b_ref, a1sa_ref, a1sb_ref, a1da_ref,
                 a1db_ref, h1a_ref, h1b_ref, as1_ref, ad1_ref):
    x = x_ref[...]
    dn = (((1,), (1,)), ((), ()))
    ha = lax.dot_general(x, w1a_ref[...], dn, preferred_element_type=jnp.float32)
    hb = lax.dot_general(x, w1b_ref[...], dn, preferred_element_type=jnp.float32)
    h1a_ref[...] = ha
    h1b_ref[...] = hb
    dnv = (((1,), (0,)), ((), ()))
    as1_ref[...] = (lax.dot_general(ha, a1sa_ref[...], dnv, preferred_element_type=jnp.float32)
                    + lax.dot_general(hb, a1sb_ref[...], dnv, preferred_element_type=jnp.float32))
    ad1_ref[...] = (lax.dot_general(ha, a1da_ref[...], dnv, preferred_element_type=jnp.float32)
                    + lax.dot_general(hb, a1db_ref[...], dnv, preferred_element_type=jnp.float32))


def _rows_spec(w):
    return pl.BlockSpec((BLK, w), lambda i: (i, 0))


def _full_spec(r, c):
    return pl.BlockSpec((r, c), lambda i: (0, 0))


@jax.jit
def _dense1(x_p, W1a, W1b, a1sa, a1sb, a1da, a1db):
    return pl.pallas_call(
        _dense1_body,
        grid=(NP // BLK,),
        in_specs=[_rows_spec(128), _full_spec(128, 128), _full_spec(128, 128),
                  _full_spec(128, 1), _full_spec(128, 1),
                  _full_spec(128, 1), _full_spec(128, 1)],
        out_specs=[_rows_spec(128), _rows_spec(128), _rows_spec(1), _rows_spec(1)],
        out_shape=[jax.ShapeDtypeStruct((NP, 128), jnp.float32),
                   jax.ShapeDtypeStruct((NP, 128), jnp.float32),
                   jax.ShapeDtypeStruct((NP, 1), jnp.float32),
                   jax.ShapeDtypeStruct((NP, 1), jnp.float32)],
    )(x_p, W1a, W1b, a1sa, a1sb, a1da, a1db)


def _dense2_body(h1a_ref, h1b_ref, as1_ref, ad1_ref,
                 na0_ref, na1_ref, nb0_ref, nb1_ref, d0_ref, d1_ref,
                 b1a_ref, b1b_ref, w2a_ref, w2b_ref, a2s_ref, a2d_ref,
                 h2_ref, as2_ref, ad2_ref):
    e = as1_ref[...] + ad1_ref[...]
    wself = jnp.exp(jnp.maximum(e, 0.0) + 0.2 * jnp.minimum(e, 0.0))
    den = d0_ref[...] + d1_ref[...] + wself + 1e-16
    outa = jnp.maximum(
        (na0_ref[...] + na1_ref[...] + wself * h1a_ref[...]) / den + b1a_ref[...], 0.0)
    outb = jnp.maximum(
        (nb0_ref[...] + nb1_ref[...] + wself * h1b_ref[...]) / den + b1b_ref[...], 0.0)
    dn = (((1,), (1,)), ((), ()))
    h2 = (lax.dot_general(outa, w2a_ref[...], dn, preferred_element_type=jnp.float32)
          + lax.dot_general(outb, w2b_ref[...], dn, preferred_element_type=jnp.float32))
    h2_ref[...] = h2
    dnv = (((1,), (0,)), ((), ()))
    as2_ref[...] = lax.dot_general(h2, a2s_ref[...], dnv, preferred_element_type=jnp.float32)
    ad2_ref[...] = lax.dot_general(h2, a2d_ref[...], dnv, preferred_element_type=jnp.float32)


@jax.jit
def _dense2(h1a, h1b, as1, ad1, na0, na1, nb0, nb1, d0, d1,
            b1a, b1b, W2a, W2b, a2s, a2d):
    return pl.pallas_call(
        _dense2_body,
        grid=(NP // BLK,),
        in_specs=[_rows_spec(128), _rows_spec(128), _rows_spec(1), _rows_spec(1),
                  _rows_spec(128), _rows_spec(128), _rows_spec(128), _rows_spec(128),
                  _rows_spec(1), _rows_spec(1),
                  _full_spec(1, 128), _full_spec(1, 128),
                  _full_spec(128, 128), _full_spec(128, 128),
                  _full_spec(128, 1), _full_spec(128, 1)],
        out_specs=[_rows_spec(128), _rows_spec(1), _rows_spec(1)],
        out_shape=[jax.ShapeDtypeStruct((NP, 128), jnp.float32),
                   jax.ShapeDtypeStruct((NP, 1), jnp.float32),
                   jax.ShapeDtypeStruct((NP, 1), jnp.float32)],
    )(h1a, h1b, as1, ad1, na0, na1, nb0, nb1, d0, d1, b1a, b1b, W2a, W2b, a2s, a2d)


def _dense3_body(h2_ref, as2_ref, ad2_ref, n0_ref, n1_ref, d0_ref, d1_ref,
                 b2_ref, wl1_ref, bl1_ref, wl2_ref, bl2_ref, y_ref):
    e = as2_ref[...] + ad2_ref[...]
    wself = jnp.exp(jnp.maximum(e, 0.0) + 0.2 * jnp.minimum(e, 0.0))
    den = d0_ref[...] + d1_ref[...] + wself + 1e-16
    out2 = jnp.maximum(
        (n0_ref[...] + n1_ref[...] + wself * h2_ref[...]) / den + b2_ref[...], 0.0)
    dn = (((1,), (1,)), ((), ()))
    t = jnp.maximum(
        lax.dot_general(out2, wl1_ref[...], dn, preferred_element_type=jnp.float32)
        + bl1_ref[...], 0.0)
    y_ref[...] = (lax.dot_general(t, wl2_ref[...], dn, preferred_element_type=jnp.float32)
                  + bl2_ref[...])


@jax.jit
def _dense3(h2, as2, ad2, n0, n1, d0, d1, b2r, Wl1, bl1r, Wl2, bl2r):
    return pl.pallas_call(
        _dense3_body,
        grid=(NP // BLK,),
        in_specs=[_rows_spec(128), _rows_spec(1), _rows_spec(1),
                  _rows_spec(128), _rows_spec(128), _rows_spec(1), _rows_spec(1),
                  _full_spec(1, 128), _full_spec(256, 128), _full_spec(1, 256),
                  _full_spec(128, 256), _full_spec(1, 128)],
        out_specs=[_rows_spec(128)],
        out_shape=[jax.ShapeDtypeStruct((NP, 128), jnp.float32)],
    )(h2, as2, ad2, n0, n1, d0, d1, b2r, Wl1, bl1r, Wl2, bl2r)[0]


def kernel(x, edge_index, batch, W1, a_src1, a_dst1, b1,
           W2, a_src2, a_dst2, b2, Wl1, bl1, Wl2, bl2):
    del batch
    x_p = jnp.pad(x, ((0, NP - NN), (0, 0)))
    pad_e = EP - EE
    src = jnp.concatenate([edge_index[0], jnp.zeros((pad_e,), jnp.int32)])
    dst = jnp.concatenate([edge_index[1], jnp.full((pad_e,), DUMP, jnp.int32)])

    W1a, W1b = W1[:128], W1[128:]
    h1a, h1b, as1, ad1 = _dense1(
        x_p, W1a, W1b,
        a_src1[:128].reshape(128, 1), a_src1[128:].reshape(128, 1),
        a_dst1[:128].reshape(128, 1), a_dst1[128:].reshape(128, 1))

    as1v = as1.reshape(NP)
    ad1v = ad1.reshape(NP)
    numA, numB, denA = _sc_conv1(src, dst, as1v, ad1v, h1a, h1b)

    h2, as2, ad2 = _dense2(
        h1a, h1b, as1, ad1,
        numA[0], numA[1], numB[0], numB[1],
        denA[0].reshape(NP, 1), denA[1].reshape(NP, 1),
        b1[:128].reshape(1, 128), b1[128:].reshape(1, 128),
        W2[:, :128], W2[:, 128:],
        a_src2.reshape(128, 1), a_dst2.reshape(128, 1))

    num2, den2 = _sc_conv2(src, dst, as2.reshape(NP), ad2.reshape(NP), h2)

    y = _dense3(
        h2, as2, ad2, num2[0], num2[1],
        den2[0].reshape(NP, 1), den2[1].reshape(NP, 1),
        b2.reshape(1, 128), Wl1, bl1.reshape(1, 256), Wl2, bl2.reshape(1, 128))
    return y[:NN]


# speedup vs baseline: 1.1352x; 1.1352x over previous
import functools

import jax
import jax.numpy as jnp
from jax import lax
from jax.experimental import pallas as pl
from jax.experimental.pallas import tpu as pltpu
from jax.experimental.pallas import tpu_sc as plsc

NN = 10000
NP = 10240
EE = 320000
CH = 128
NCHUNK = 80
EPT = CH * NCHUNK
NWORK = 32
EP = EPT * NWORK
DUMP = 10200
BLK = 1024


def _zero_tile_slice(grows0, w0, num_sh, den_sh, sid, zero_den):
    zeros16 = jnp.zeros((16,), jnp.float32)

    def zrow(r, c):
        for cc in range(8):
            grows0[r, pl.ds(cc * 16, 16)] = zeros16
        return c

    lax.fori_loop(0, CH, zrow, 0)
    for cc in range(8):
        w0[pl.ds(cc * 16, 16)] = zeros16

    r0 = sid * 640
    for k in range(5):
        pltpu.sync_copy(grows0, num_sh.at[pl.ds(r0 + k * 128, 128)])
        if zero_den:
            pltpu.sync_copy(w0, den_sh.at[pl.ds(r0 + k * 128, 128)])


def _edge_pass(eidx_hbm, asrc_v, adst_v, h_hbm, n_out, den_out,
               ebuf, w0, grows0, gsem, num_sh, den_sh, cid, sid, do_den):
    cbase = (cid * 16 + sid) * NCHUNK

    def chunk(j, carry):
        pltpu.sync_copy(eidx_hbm.at[cbase + j], ebuf)
        pltpu.async_copy(h_hbm.at[ebuf.at[0]], grows0, gsem)
        for t in range(CH // 16):
            si = ebuf[0, pl.ds(t * 16, 16)]
            di = ebuf[1, pl.ds(t * 16, 16)]
            e = plsc.load_gather(asrc_v, [si]) + plsc.load_gather(adst_v, [di])
            e = jnp.maximum(e, 0.0) + 0.2 * jnp.minimum(e, 0.0)
            w0[pl.ds(t * 16, 16)] = jnp.exp(e)
        pltpu.make_async_copy(h_hbm.at[ebuf.at[0]], grows0, gsem).wait()

        def scale(r, carry2):
            rr = 2 * r
            wa = plsc.load_gather(w0, [jnp.zeros((16,), jnp.int32) + rr])
            wb = plsc.load_gather(w0, [jnp.zeros((16,), jnp.int32) + (rr + 1)])
            for cc in range(8):
                grows0[rr, pl.ds(cc * 16, 16)] = grows0[rr, pl.ds(cc * 16, 16)] * wa
            for cc in range(8):
                grows0[rr + 1, pl.ds(cc * 16, 16)] = grows0[rr + 1, pl.ds(cc * 16, 16)] * wb
            return carry2

        lax.fori_loop(0, CH // 2, scale, 0)
        pltpu.sync_copy(grows0, num_sh.at[ebuf.at[1]], add=True)
        if do_den:
            pltpu.sync_copy(w0, den_sh.at[ebuf.at[1]], add=True)
        return carry

    lax.fori_loop(0, NCHUNK, chunk, 0)
    plsc.subcore_barrier()

    r0 = sid * 640
    for k in range(5):
        pltpu.sync_copy(num_sh.at[pl.ds(r0 + k * 128, 128)],
                        n_out.at[cid, pl.ds(r0 + k * 128, 128)])
        if do_den:
            pltpu.sync_copy(den_sh.at[pl.ds(r0 + k * 128, 128)],
                            den_out.at[cid, pl.ds(r0 + k * 128, 128)])


def _sc_conv1_body(eidx_hbm, asrc_hbm, adst_hbm, ha_hbm, hb_hbm,
                   numa_out, numb_out, den_out,
                   asrc_v, adst_v, ebuf, w0, grows0, gsem,
                   num_sh, den_sh):
    cid = lax.axis_index("c")
    sid = lax.axis_index("s")

    _zero_tile_slice(grows0, w0, num_sh, den_sh, sid, zero_den=True)
    pltpu.sync_copy(asrc_hbm, asrc_v)
    pltpu.sync_copy(adst_hbm, adst_v)
    plsc.subcore_barrier()

    _edge_pass(eidx_hbm, asrc_v, adst_v, ha_hbm, numa_out, den_out,
               ebuf, w0, grows0, gsem, num_sh, den_sh, cid, sid, do_den=True)
    _zero_tile_slice(grows0, w0, num_sh, den_sh, sid, zero_den=False)
    plsc.subcore_barrier()
    _edge_pass(eidx_hbm, asrc_v, adst_v, hb_hbm, numb_out, den_out,
               ebuf, w0, grows0, gsem, num_sh, den_sh, cid, sid, do_den=False)


def _sc_conv2_body(eidx_hbm, asrc_hbm, adst_hbm, h_hbm,
                   num_out, den_out,
                   asrc_v, adst_v, ebuf, w0, grows0, gsem,
                   num_sh, den_sh):
    cid = lax.axis_index("c")
    sid = lax.axis_index("s")

    _zero_tile_slice(grows0, w0, num_sh, den_sh, sid, zero_den=True)
    pltpu.sync_copy(asrc_hbm, asrc_v)
    pltpu.sync_copy(adst_hbm, adst_v)
    plsc.subcore_barrier()
    _edge_pass(eidx_hbm, asrc_v, adst_v, h_hbm, num_out, den_out,
               ebuf, w0, grows0, gsem, num_sh, den_sh, cid, sid, do_den=True)


def _sc_scratch():
    return [
            pltpu.VMEM((NP,), jnp.float32),
            pltpu.VMEM((NP,), jnp.float32),
            pltpu.VMEM((2, CH), jnp.int32),
            pltpu.VMEM((CH,), jnp.float32),
            pltpu.VMEM((CH, 128), jnp.float32),
            pltpu.SemaphoreType.DMA,
            pltpu.VMEM_SHARED((NP, 128), jnp.float32),
            pltpu.VMEM_SHARED((NP,), jnp.float32),
    ]


@jax.jit
def _sc_conv1(eidx, a_s, a_d, ha, hb):
    mesh = plsc.VectorSubcoreMesh(core_axis_name="c", subcore_axis_name="s")
    f = pl.kernel(
        _sc_conv1_body,
        mesh=mesh,
        compiler_params=pltpu.CompilerParams(needs_layout_passes=False),
        out_type=[
            jax.ShapeDtypeStruct((2, NP, 128), jnp.float32),
            jax.ShapeDtypeStruct((2, NP, 128), jnp.float32),
            jax.ShapeDtypeStruct((2, NP), jnp.float32),
        ],
        scratch_types=_sc_scratch(),
    )
    return f(eidx, a_s, a_d, ha, hb)


@jax.jit
def _sc_conv2(eidx, a_s, a_d, h):
    mesh = plsc.VectorSubcoreMesh(core_axis_name="c", subcore_axis_name="s")
    f = pl.kernel(
        _sc_conv2_body,
        mesh=mesh,
        compiler_params=pltpu.CompilerParams(needs_layout_passes=False),
        out_type=[
            jax.ShapeDtypeStruct((2, NP, 128), jnp.float32),
            jax.ShapeDtypeStruct((2, NP), jnp.float32),
        ],
        scratch_types=_sc_scratch(),
    )
    return f(eidx, a_s, a_d, h)


def _dense1_body(x_ref, w1a_ref, w1b_ref, a1sa_ref, a1sb_ref, a1da_ref,
                 a1db_ref, h1a_ref, h1b_ref, as1_ref, ad1_ref):
    x = x_ref[...]
    dn = (((1,), (1,)), ((), ()))
    ha = lax.dot_general(x, w1a_ref[...], dn, preferred_element_type=jnp.float32)
    hb = lax.dot_general(x, w1b_ref[...], dn, preferred_element_type=jnp.float32)
    h1a_ref[...] = ha
    h1b_ref[...] = hb
    dnv = (((1,), (0,)), ((), ()))
    as1_ref[...] = (lax.dot_general(ha, a1sa_ref[...], dnv, preferred_element_type=jnp.float32)
                    + lax.dot_general(hb, a1sb_ref[...], dnv, preferred_element_type=jnp.float32))
    ad1_ref[...] = (lax.dot_general(ha, a1da_ref[...], dnv, preferred_element_type=jnp.float32)
                    + lax.dot_general(hb, a1db_ref[...], dnv, preferred_element_type=jnp.float32))


def _rows_spec(w):
    return pl.BlockSpec((BLK, w), lambda i: (i, 0))


def _full_spec(r, c):
    return pl.BlockSpec((r, c), lambda i: (0, 0))


@jax.jit
def _dense1(x_p, W1a, W1b, a1sa, a1sb, a1da, a1db):
    return pl.pallas_call(
        _dense1_body,
        grid=(NP // BLK,),
        in_specs=[_rows_spec(128), _full_spec(128, 128), _full_spec(128, 128),
                  _full_spec(128, 1), _full_spec(128, 1),
                  _full_spec(128, 1), _full_spec(128, 1)],
        out_specs=[_rows_spec(128), _rows_spec(128), _rows_spec(1), _rows_spec(1)],
        out_shape=[jax.ShapeDtypeStruct((NP, 128), jnp.float32),
                   jax.ShapeDtypeStruct((NP, 128), jnp.float32),
                   jax.ShapeDtypeStruct((NP, 1), jnp.float32),
                   jax.ShapeDtypeStruct((NP, 1), jnp.float32)],
    )(x_p, W1a, W1b, a1sa, a1sb, a1da, a1db)


def _dense2_body(h1a_ref, h1b_ref, as1_ref, ad1_ref,
                 na0_ref, na1_ref, nb0_ref, nb1_ref, d0_ref, d1_ref,
                 b1a_ref, b1b_ref, w2a_ref, w2b_ref, a2s_ref, a2d_ref,
                 h2_ref, as2_ref, ad2_ref):
    e = as1_ref[...] + ad1_ref[...]
    wself = jnp.exp(jnp.maximum(e, 0.0) + 0.2 * jnp.minimum(e, 0.0))
    den = d0_ref[...] + d1_ref[...] + wself + 1e-16
    outa = jnp.maximum(
        (na0_ref[...] + na1_ref[...] + wself * h1a_ref[...]) / den + b1a_ref[...], 0.0)
    outb = jnp.maximum(
        (nb0_ref[...] + nb1_ref[...] + wself * h1b_ref[...]) / den + b1b_ref[...], 0.0)
    dn = (((1,), (1,)), ((), ()))
    h2 = (lax.dot_general(outa, w2a_ref[...], dn, preferred_element_type=jnp.float32)
          + lax.dot_general(outb, w2b_ref[...], dn, preferred_element_type=jnp.float32))
    h2_ref[...] = h2
    dnv = (((1,), (0,)), ((), ()))
    as2_ref[...] = lax.dot_general(h2, a2s_ref[...], dnv, preferred_element_type=jnp.float32)
    ad2_ref[...] = lax.dot_general(h2, a2d_ref[...], dnv, preferred_element_type=jnp.float32)


@jax.jit
def _dense2(h1a, h1b, as1, ad1, na0, na1, nb0, nb1, d0, d1,
            b1a, b1b, W2a, W2b, a2s, a2d):
    return pl.pallas_call(
        _dense2_body,
        grid=(NP // BLK,),
        in_specs=[_rows_spec(128), _rows_spec(128), _rows_spec(1), _rows_spec(1),
                  _rows_spec(128), _rows_spec(128), _rows_spec(128), _rows_spec(128),
                  _rows_spec(1), _rows_spec(1),
                  _full_spec(1, 128), _full_spec(1, 128),
                  _full_spec(128, 128), _full_spec(128, 128),
                  _full_spec(128, 1), _full_spec(128, 1)],
        out_specs=[_rows_spec(128), _rows_spec(1), _rows_spec(1)],
        out_shape=[jax.ShapeDtypeStruct((NP, 128), jnp.float32),
                   jax.ShapeDtypeStruct((NP, 1), jnp.float32),
                   jax.ShapeDtypeStruct((NP, 1), jnp.float32)],
    )(h1a, h1b, as1, ad1, na0, na1, nb0, nb1, d0, d1, b1a, b1b, W2a, W2b, a2s, a2d)


def _dense3_body(h2_ref, as2_ref, ad2_ref, n0_ref, n1_ref, d0_ref, d1_ref,
                 b2_ref, wl1_ref, bl1_ref, wl2_ref, bl2_ref, y_ref):
    e = as2_ref[...] + ad2_ref[...]
    wself = jnp.exp(jnp.maximum(e, 0.0) + 0.2 * jnp.minimum(e, 0.0))
    den = d0_ref[...] + d1_ref[...] + wself + 1e-16
    out2 = jnp.maximum(
        (n0_ref[...] + n1_ref[...] + wself * h2_ref[...]) / den + b2_ref[...], 0.0)
    dn = (((1,), (1,)), ((), ()))
    t = jnp.maximum(
        lax.dot_general(out2, wl1_ref[...], dn, preferred_element_type=jnp.float32)
        + bl1_ref[...], 0.0)
    y_ref[...] = (lax.dot_general(t, wl2_ref[...], dn, preferred_element_type=jnp.float32)
                  + bl2_ref[...])


@jax.jit
def _dense3(h2, as2, ad2, n0, n1, d0, d1, b2r, Wl1, bl1r, Wl2, bl2r):
    return pl.pallas_call(
        _dense3_body,
        grid=(NP // BLK,),
        in_specs=[_rows_spec(128), _rows_spec(1), _rows_spec(1),
                  _rows_spec(128), _rows_spec(128), _rows_spec(1), _rows_spec(1),
                  _full_spec(1, 128), _full_spec(256, 128), _full_spec(1, 256),
                  _full_spec(128, 256), _full_spec(1, 128)],
        out_specs=[_rows_spec(128)],
        out_shape=[jax.ShapeDtypeStruct((NP, 128), jnp.float32)],
    )(h2, as2, ad2, n0, n1, d0, d1, b2r, Wl1, bl1r, Wl2, bl2r)[0]


def kernel(x, edge_index, batch, W1, a_src1, a_dst1, b1,
           W2, a_src2, a_dst2, b2, Wl1, bl1, Wl2, bl2):
    del batch
    x_p = jnp.pad(x, ((0, NP - NN), (0, 0)))
    pad_e = EP - EE
    src = jnp.concatenate([edge_index[0], jnp.zeros((pad_e,), jnp.int32)])
    dst = jnp.concatenate([edge_index[1], jnp.full((pad_e,), DUMP, jnp.int32)])
    eidx = jnp.stack([src.reshape(-1, CH), dst.reshape(-1, CH)], axis=1)

    W1a, W1b = W1[:128], W1[128:]
    h1a, h1b, as1, ad1 = _dense1(
        x_p, W1a, W1b,
        a_src1[:128].reshape(128, 1), a_src1[128:].reshape(128, 1),
        a_dst1[:128].reshape(128, 1), a_dst1[128:].reshape(128, 1))

    as1v = as1.reshape(NP)
    ad1v = ad1.reshape(NP)
    numA, numB, denA = _sc_conv1(eidx, as1v, ad1v, h1a, h1b)

    h2, as2, ad2 = _dense2(
        h1a, h1b, as1, ad1,
        numA[0], numA[1], numB[0], numB[1],
        denA[0].reshape(NP, 1), denA[1].reshape(NP, 1),
        b1[:128].reshape(1, 128), b1[128:].reshape(1, 128),
        W2[:, :128], W2[:, 128:],
        a_src2.reshape(128, 1), a_dst2.reshape(128, 1))

    num2, den2 = _sc_conv2(eidx, as2.reshape(NP), ad2.reshape(NP), h2)

    y = _dense3(
        h2, as2, ad2, num2[0], num2[1],
        den2[0].reshape(NP, 1), den2[1].reshape(NP, 1),
        b2.reshape(1, 128), Wl1, bl1.reshape(1, 256), Wl2, bl2.reshape(1, 128))
    return y[:NN]


# speedup vs baseline: 1.1786x; 1.0382x over previous
import functools

import jax
import jax.numpy as jnp
from jax import lax
from jax.experimental import pallas as pl
from jax.experimental.pallas import tpu as pltpu
from jax.experimental.pallas import tpu_sc as plsc

NN = 10000
NP = 10240
EE = 320000
CH = 128
NCHUNK = 80
EPT = CH * NCHUNK
NWORK = 32
EP = EPT * NWORK
DUMP = 10200
BLK = 1024


def _zero_tile_slice(grows0, w0, num_sh, den_sh, sid, zero_den):
    zeros16 = jnp.zeros((16,), jnp.float32)

    def zrow(r, c):
        for cc in range(8):
            grows0[r, pl.ds(cc * 16, 16)] = zeros16
        return c

    lax.fori_loop(0, CH, zrow, 0)
    for cc in range(8):
        w0[pl.ds(cc * 16, 16)] = zeros16

    r0 = sid * 640
    for k in range(5):
        pltpu.sync_copy(grows0, num_sh.at[pl.ds(r0 + k * 128, 128)])
        if zero_den:
            pltpu.sync_copy(w0, den_sh.at[pl.ds(r0 + k * 128, 128)])


def _edge_pass(eidx_hbm, asrc_v, adst_v, h_hbm, n_out, den_out,
               ebuf, w0, grows0, gsem, num_sh, den_sh, cid, sid, do_den):
    cbase = (cid * 16 + sid) * NCHUNK

    def chunk(j, carry):
        pltpu.sync_copy(eidx_hbm.at[cbase + j], ebuf)
        pltpu.async_copy(h_hbm.at[ebuf.at[0]], grows0, gsem)
        for t in range(CH // 16):
            si = ebuf[0, pl.ds(t * 16, 16)]
            di = ebuf[1, pl.ds(t * 16, 16)]
            e = plsc.load_gather(asrc_v, [si]) + plsc.load_gather(adst_v, [di])
            e = jnp.maximum(e, 0.0) + 0.2 * jnp.minimum(e, 0.0)
            w0[pl.ds(t * 16, 16)] = jnp.exp(e)
        pltpu.make_async_copy(h_hbm.at[ebuf.at[0]], grows0, gsem).wait()

        def scale(r, carry2):
            rr = 2 * r
            wa = plsc.load_gather(w0, [jnp.zeros((16,), jnp.int32) + rr])
            wb = plsc.load_gather(w0, [jnp.zeros((16,), jnp.int32) + (rr + 1)])
            for cc in range(8):
                grows0[rr, pl.ds(cc * 16, 16)] = grows0[rr, pl.ds(cc * 16, 16)] * wa
            for cc in range(8):
                grows0[rr + 1, pl.ds(cc * 16, 16)] = grows0[rr + 1, pl.ds(cc * 16, 16)] * wb
            return carry2

        lax.fori_loop(0, CH // 2, scale, 0)
        pltpu.sync_copy(grows0, num_sh.at[ebuf.at[1]], add=True)
        if do_den:
            pltpu.sync_copy(w0, den_sh.at[ebuf.at[1]], add=True)
        return carry

    lax.fori_loop(0, NCHUNK, chunk, 0)
    plsc.subcore_barrier()

    r0 = sid * 640
    for k in range(5):
        pltpu.sync_copy(num_sh.at[pl.ds(r0 + k * 128, 128)],
                        n_out.at[cid, pl.ds(r0 + k * 128, 128)])
        if do_den:
            pltpu.sync_copy(den_sh.at[pl.ds(r0 + k * 128, 128)],
                            den_out.at[cid, pl.ds(r0 + k * 128, 128)])


def _sc_agg_body(eidx_hbm, asrc_hbm, adst_hbm, h_hbm,
                   num_out, den_out,
                   asrc_v, adst_v, ebuf, w0, grows0, gsem,
                   num_sh, den_sh):
    cid = lax.axis_index("c")
    sid = lax.axis_index("s")

    _zero_tile_slice(grows0, w0, num_sh, den_sh, sid, zero_den=True)
    pltpu.sync_copy(asrc_hbm, asrc_v)
    pltpu.sync_copy(adst_hbm, adst_v)
    plsc.subcore_barrier()
    _edge_pass(eidx_hbm, asrc_v, adst_v, h_hbm, num_out, den_out,
               ebuf, w0, grows0, gsem, num_sh, den_sh, cid, sid, do_den=True)


def _sc_scratch():
    return [
            pltpu.VMEM((NP,), jnp.float32),
            pltpu.VMEM((NP,), jnp.float32),
            pltpu.VMEM((2, CH), jnp.int32),
            pltpu.VMEM((CH,), jnp.float32),
            pltpu.VMEM((CH, 128), jnp.float32),
            pltpu.SemaphoreType.DMA,
            pltpu.VMEM_SHARED((NP, 128), jnp.float32),
            pltpu.VMEM_SHARED((NP,), jnp.float32),
    ]


@jax.jit
def _sc_agg(eidx, a_s, a_d, h):
    mesh = plsc.VectorSubcoreMesh(core_axis_name="c", subcore_axis_name="s")
    f = pl.kernel(
        _sc_agg_body,
        mesh=mesh,
        compiler_params=pltpu.CompilerParams(needs_layout_passes=False),
        out_type=[
            jax.ShapeDtypeStruct((2, NP, 128), jnp.float32),
            jax.ShapeDtypeStruct((2, NP), jnp.float32),
        ],
        scratch_types=_sc_scratch(),
    )
    return f(eidx, a_s, a_d, h)


def _dense1_body(x_ref, w1a_ref, w1b_ref, a1sa_ref, a1sb_ref, a1da_ref,
                 a1db_ref, h1a_ref, h1b_ref, as1_ref, ad1_ref):
    x = x_ref[...]
    dn = (((1,), (1,)), ((), ()))
    ha = lax.dot_general(x, w1a_ref[...], dn, preferred_element_type=jnp.float32)
    hb = lax.dot_general(x, w1b_ref[...], dn, preferred_element_type=jnp.float32)
    h1a_ref[...] = ha
    h1b_ref[...] = hb
    dnv = (((1,), (0,)), ((), ()))
    as1_ref[...] = (lax.dot_general(ha, a1sa_ref[...], dnv, preferred_element_type=jnp.float32)
                    + lax.dot_general(hb, a1sb_ref[...], dnv, preferred_element_type=jnp.float32))
    ad1_ref[...] = (lax.dot_general(ha, a1da_ref[...], dnv, preferred_element_type=jnp.float32)
                    + lax.dot_general(hb, a1db_ref[...], dnv, preferred_element_type=jnp.float32))


def _rows_spec(w):
    return pl.BlockSpec((BLK, w), lambda i: (i, 0))


def _full_spec(r, c):
    return pl.BlockSpec((r, c), lambda i: (0, 0))


@jax.jit
def _dense1(x_p, W1a, W1b, a1sa, a1sb, a1da, a1db):
    return pl.pallas_call(
        _dense1_body,
        grid=(NP // BLK,),
        in_specs=[_rows_spec(128), _full_spec(128, 128), _full_spec(128, 128),
                  _full_spec(128, 1), _full_spec(128, 1),
                  _full_spec(128, 1), _full_spec(128, 1)],
        out_specs=[_rows_spec(128), _rows_spec(128), _rows_spec(1), _rows_spec(1)],
        out_shape=[jax.ShapeDtypeStruct((NP, 128), jnp.float32),
                   jax.ShapeDtypeStruct((NP, 128), jnp.float32),
                   jax.ShapeDtypeStruct((NP, 1), jnp.float32),
                   jax.ShapeDtypeStruct((NP, 1), jnp.float32)],
    )(x_p, W1a, W1b, a1sa, a1sb, a1da, a1db)


def _dense2_body(h1a_ref, h1b_ref, as1_ref, ad1_ref,
                 na0_ref, na1_ref, nb0_ref, nb1_ref, d0_ref, d1_ref,
                 b1a_ref, b1b_ref, w2a_ref, w2b_ref, a2s_ref, a2d_ref,
                 h2_ref, as2_ref, ad2_ref):
    e = as1_ref[...] + ad1_ref[...]
    wself = jnp.exp(jnp.maximum(e, 0.0) + 0.2 * jnp.minimum(e, 0.0))
    den = d0_ref[...] + d1_ref[...] + wself + 1e-16
    outa = jnp.maximum(
        (na0_ref[...] + na1_ref[...] + wself * h1a_ref[...]) / den + b1a_ref[...], 0.0)
    outb = jnp.maximum(
        (nb0_ref[...] + nb1_ref[...] + wself * h1b_ref[...]) / den + b1b_ref[...], 0.0)
    dn = (((1,), (1,)), ((), ()))
    h2 = (lax.dot_general(outa, w2a_ref[...], dn, preferred_element_type=jnp.float32)
          + lax.dot_general(outb, w2b_ref[...], dn, preferred_element_type=jnp.float32))
    h2_ref[...] = h2
    dnv = (((1,), (0,)), ((), ()))
    as2_ref[...] = lax.dot_general(h2, a2s_ref[...], dnv, preferred_element_type=jnp.float32)
    ad2_ref[...] = lax.dot_general(h2, a2d_ref[...], dnv, preferred_element_type=jnp.float32)


@jax.jit
def _dense2(h1a, h1b, as1, ad1, na0, na1, nb0, nb1, d0, d1,
            b1a, b1b, W2a, W2b, a2s, a2d):
    return pl.pallas_call(
        _dense2_body,
        grid=(NP // BLK,),
        in_specs=[_rows_spec(128), _rows_spec(128), _rows_spec(1), _rows_spec(1),
                  _rows_spec(128), _rows_spec(128), _rows_spec(128), _rows_spec(128),
                  _rows_spec(1), _rows_spec(1),
                  _full_spec(1, 128), _full_spec(1, 128),
                  _full_spec(128, 128), _full_spec(128, 128),
                  _full_spec(128, 1), _full_spec(128, 1)],
        out_specs=[_rows_spec(128), _rows_spec(1), _rows_spec(1)],
        out_shape=[jax.ShapeDtypeStruct((NP, 128), jnp.float32),
                   jax.ShapeDtypeStruct((NP, 1), jnp.float32),
                   jax.ShapeDtypeStruct((NP, 1), jnp.float32)],
    )(h1a, h1b, as1, ad1, na0, na1, nb0, nb1, d0, d1, b1a, b1b, W2a, W2b, a2s, a2d)


def _dense3_body(h2_ref, as2_ref, ad2_ref, n0_ref, n1_ref, d0_ref, d1_ref,
                 b2_ref, wl1_ref, bl1_ref, wl2_ref, bl2_ref, y_ref):
    e = as2_ref[...] + ad2_ref[...]
    wself = jnp.exp(jnp.maximum(e, 0.0) + 0.2 * jnp.minimum(e, 0.0))
    den = d0_ref[...] + d1_ref[...] + wself + 1e-16
    out2 = jnp.maximum(
        (n0_ref[...] + n1_ref[...] + wself * h2_ref[...]) / den + b2_ref[...], 0.0)
    dn = (((1,), (1,)), ((), ()))
    t = jnp.maximum(
        lax.dot_general(out2, wl1_ref[...], dn, preferred_element_type=jnp.float32)
        + bl1_ref[...], 0.0)
    y_ref[...] = (lax.dot_general(t, wl2_ref[...], dn, preferred_element_type=jnp.float32)
                  + bl2_ref[...])


@jax.jit
def _dense3(h2, as2, ad2, n0, n1, d0, d1, b2r, Wl1, bl1r, Wl2, bl2r):
    return pl.pallas_call(
        _dense3_body,
        grid=(NP // BLK,),
        in_specs=[_rows_spec(128), _rows_spec(1), _rows_spec(1),
                  _rows_spec(128), _rows_spec(128), _rows_spec(1), _rows_spec(1),
                  _full_spec(1, 128), _full_spec(256, 128), _full_spec(1, 256),
                  _full_spec(128, 256), _full_spec(1, 128)],
        out_specs=[_rows_spec(128)],
        out_shape=[jax.ShapeDtypeStruct((NP, 128), jnp.float32)],
    )(h2, as2, ad2, n0, n1, d0, d1, b2r, Wl1, bl1r, Wl2, bl2r)[0]


def kernel(x, edge_index, batch, W1, a_src1, a_dst1, b1,
           W2, a_src2, a_dst2, b2, Wl1, bl1, Wl2, bl2):
    del batch
    x_p = jnp.pad(x, ((0, NP - NN), (0, 0)))
    pad_e = EP - EE
    src = jnp.concatenate([edge_index[0], jnp.zeros((pad_e,), jnp.int32)])
    dst = jnp.concatenate([edge_index[1], jnp.full((pad_e,), DUMP, jnp.int32)])
    eidx = jnp.stack([src.reshape(-1, CH), dst.reshape(-1, CH)], axis=1)

    W1a, W1b = W1[:128], W1[128:]
    h1a, h1b, as1, ad1 = _dense1(
        x_p, W1a, W1b,
        a_src1[:128].reshape(128, 1), a_src1[128:].reshape(128, 1),
        a_dst1[:128].reshape(128, 1), a_dst1[128:].reshape(128, 1))

    as1v = as1.reshape(NP)
    ad1v = ad1.reshape(NP)
    numA, denA = _sc_agg(eidx, as1v, ad1v, h1a)
    numB, _ = _sc_agg(eidx, as1v, ad1v, h1b)

    h2, as2, ad2 = _dense2(
        h1a, h1b, as1, ad1,
        numA[0], numA[1], numB[0], numB[1],
        denA[0].reshape(NP, 1), denA[1].reshape(NP, 1),
        b1[:128].reshape(1, 128), b1[128:].reshape(1, 128),
        W2[:, :128], W2[:, 128:],
        a_src2.reshape(128, 1), a_dst2.reshape(128, 1))

    num2, den2 = _sc_agg(eidx, as2.reshape(NP), ad2.reshape(NP), h2)

    y = _dense3(
        h2, as2, ad2, num2[0], num2[1],
        den2[0].reshape(NP, 1), den2[1].reshape(NP, 1),
        b2.reshape(1, 128), Wl1, bl1.reshape(1, 256), Wl2, bl2.reshape(1, 128))
    return y[:NN]


# speedup vs baseline: 1.3973x; 1.1856x over previous
import functools

import jax
import jax.numpy as jnp
from jax import lax
from jax.experimental import pallas as pl
from jax.experimental.pallas import tpu as pltpu
from jax.experimental.pallas import tpu_sc as plsc

NN = 10000
NP = 10240
EE = 320000
CH = 128
NCHUNK = 79
EPT = CH * NCHUNK
NWORK = 32
EP = EPT * NWORK
DUMP = 10200
BLK = 1024


def _sc_agg_body(src_hbm, dst_hbm, asrc_hbm, adst_hbm, h_hbm,
                 num_out, den_out,
                 asrc_v, adst_v, sidx_v, didx_v, w_v, rows_v, sem,
                 num_sh, den_sh):
    cid = lax.axis_index("c")
    sid = lax.axis_index("s")

    zeros16 = jnp.zeros((16,), jnp.float32)

    def zrow(r, c):
        for cc in range(8):
            rows_v[r, pl.ds(cc * 16, 16)] = zeros16
        return c

    lax.fori_loop(0, CH, zrow, 0)
    for cc in range(8):
        w_v[pl.ds(cc * 16, 16)] = zeros16

    r0 = sid * 640
    for k in range(5):
        pltpu.sync_copy(rows_v, num_sh.at[pl.ds(r0 + k * 128, 128)])
        pltpu.sync_copy(w_v, den_sh.at[pl.ds(r0 + k * 128, 128)])

    pltpu.sync_copy(asrc_hbm, asrc_v)
    pltpu.sync_copy(adst_hbm, adst_v)
    plsc.subcore_barrier()

    wid = cid * 16 + sid
    ebase = wid * EPT

    def chunk(j, carry):
        eb = ebase + j * CH
        pltpu.sync_copy(src_hbm.at[pl.ds(eb, CH)], sidx_v)
        pltpu.sync_copy(dst_hbm.at[pl.ds(eb, CH)], didx_v)
        for t in range(8):
            si = sidx_v[pl.ds(t * 16, 16)]
            di = didx_v[pl.ds(t * 16, 16)]
            e = plsc.load_gather(asrc_v, [si]) + plsc.load_gather(adst_v, [di])
            e = jnp.maximum(e, 0.0) + 0.2 * jnp.minimum(e, 0.0)
            w_v[pl.ds(t * 16, 16)] = jnp.exp(e)
        pltpu.async_copy(h_hbm.at[sidx_v], rows_v, sem).wait()

        def scale(r, carry2):
            wr = plsc.load_gather(w_v, [jnp.zeros((16,), jnp.int32) + r])
            for cc in range(8):
                rows_v[r, pl.ds(cc * 16, 16)] = rows_v[r, pl.ds(cc * 16, 16)] * wr
            return carry2

        lax.fori_loop(0, CH, scale, 0)
        pltpu.sync_copy(rows_v, num_sh.at[didx_v], add=True)
        pltpu.sync_copy(w_v, den_sh.at[didx_v], add=True)
        return carry

    lax.fori_loop(0, NCHUNK, chunk, 0)
    plsc.subcore_barrier()

    for k in range(5):
        pltpu.sync_copy(num_sh.at[pl.ds(r0 + k * 128, 128)],
                        num_out.at[cid, pl.ds(r0 + k * 128, 128)])
        pltpu.sync_copy(den_sh.at[pl.ds(r0 + k * 128, 128)],
                        den_out.at[cid, pl.ds(r0 + k * 128, 128)])


@jax.jit
def _sc_agg(src, dst, a_s, a_d, h):
    mesh = plsc.VectorSubcoreMesh(core_axis_name="c", subcore_axis_name="s")
    f = pl.kernel(
        _sc_agg_body,
        mesh=mesh,
        compiler_params=pltpu.CompilerParams(needs_layout_passes=False),
        out_type=[
            jax.ShapeDtypeStruct((2, NP, 128), jnp.float32),
            jax.ShapeDtypeStruct((2, NP), jnp.float32),
        ],
        scratch_types=[
            pltpu.VMEM((NP,), jnp.float32),
            pltpu.VMEM((NP,), jnp.float32),
            pltpu.VMEM((CH,), jnp.int32),
            pltpu.VMEM((CH,), jnp.int32),
            pltpu.VMEM((CH,), jnp.float32),
            pltpu.VMEM((CH, 128), jnp.float32),
            pltpu.SemaphoreType.DMA,
            pltpu.VMEM_SHARED((NP, 128), jnp.float32),
            pltpu.VMEM_SHARED((NP,), jnp.float32),
        ],
    )
    return f(src, dst, a_s, a_d, h)


def _dense1_body(x_ref, w1a_ref, w1b_ref, a1sa_ref, a1sb_ref, a1da_ref,
                 a1db_ref, h1a_ref, h1b_ref, as1_ref, ad1_ref):
    x = x_ref[...]
    dn = (((1,), (1,)), ((), ()))
    ha = lax.dot_general(x, w1a_ref[...], dn, preferred_element_type=jnp.float32)
    hb = lax.dot_general(x, w1b_ref[...], dn, preferred_element_type=jnp.float32)
    h1a_ref[...] = ha
    h1b_ref[...] = hb
    dnv = (((1,), (0,)), ((), ()))
    as1_ref[...] = (lax.dot_general(ha, a1sa_ref[...], dnv, preferred_element_type=jnp.float32)
                    + lax.dot_general(hb, a1sb_ref[...], dnv, preferred_element_type=jnp.float32))
    ad1_ref[...] = (lax.dot_general(ha, a1da_ref[...], dnv, preferred_element_type=jnp.float32)
                    + lax.dot_general(hb, a1db_ref[...], dnv, preferred_element_type=jnp.float32))


def _rows_spec(w):
    return pl.BlockSpec((BLK, w), lambda i: (i, 0))


def _full_spec(r, c):
    return pl.BlockSpec((r, c), lambda i: (0, 0))


@jax.jit
def _dense1(x_p, W1a, W1b, a1sa, a1sb, a1da, a1db):
    return pl.pallas_call(
        _dense1_body,
        grid=(NP // BLK,),
        in_specs=[_rows_spec(128), _full_spec(128, 128), _full_spec(128, 128),
                  _full_spec(128, 1), _full_spec(128, 1),
                  _full_spec(128, 1), _full_spec(128, 1)],
        out_specs=[_rows_spec(128), _rows_spec(128), _rows_spec(1), _rows_spec(1)],
        out_shape=[jax.ShapeDtypeStruct((NP, 128), jnp.float32),
                   jax.ShapeDtypeStruct((NP, 128), jnp.float32),
                   jax.ShapeDtypeStruct((NP, 1), jnp.float32),
                   jax.ShapeDtypeStruct((NP, 1), jnp.float32)],
    )(x_p, W1a, W1b, a1sa, a1sb, a1da, a1db)


def _dense2_body(h1a_ref, h1b_ref, as1_ref, ad1_ref,
                 na0_ref, na1_ref, nb0_ref, nb1_ref, d0_ref, d1_ref,
                 b1a_ref, b1b_ref, w2a_ref, w2b_ref, a2s_ref, a2d_ref,
                 h2_ref, as2_ref, ad2_ref):
    e = as1_ref[...] + ad1_ref[...]
    wself = jnp.exp(jnp.maximum(e, 0.0) + 0.2 * jnp.minimum(e, 0.0))
    den = d0_ref[...] + d1_ref[...] + wself + 1e-16
    outa = jnp.maximum(
        (na0_ref[...] + na1_ref[...] + wself * h1a_ref[...]) / den + b1a_ref[...], 0.0)
    outb = jnp.maximum(
        (nb0_ref[...] + nb1_ref[...] + wself * h1b_ref[...]) / den + b1b_ref[...], 0.0)
    dn = (((1,), (1,)), ((), ()))
    h2 = (lax.dot_general(outa, w2a_ref[...], dn, preferred_element_type=jnp.float32)
          + lax.dot_general(outb, w2b_ref[...], dn, preferred_element_type=jnp.float32))
    h2_ref[...] = h2
    dnv = (((1,), (0,)), ((), ()))
    as2_ref[...] = lax.dot_general(h2, a2s_ref[...], dnv, preferred_element_type=jnp.float32)
    ad2_ref[...] = lax.dot_general(h2, a2d_ref[...], dnv, preferred_element_type=jnp.float32)


@jax.jit
def _dense2(h1a, h1b, as1, ad1, na0, na1, nb0, nb1, d0, d1,
            b1a, b1b, W2a, W2b, a2s, a2d):
    return pl.pallas_call(
        _dense2_body,
        grid=(NP // BLK,),
        in_specs=[_rows_spec(128), _rows_spec(128), _rows_spec(1), _rows_spec(1),
                  _rows_spec(128), _rows_spec(128), _rows_spec(128), _rows_spec(128),
                  _rows_spec(1), _rows_spec(1),
                  _full_spec(1, 128), _full_spec(1, 128),
                  _full_spec(128, 128), _full_spec(128, 128),
                  _full_spec(128, 1), _full_spec(128, 1)],
        out_specs=[_rows_spec(128), _rows_spec(1), _rows_spec(1)],
        out_shape=[jax.ShapeDtypeStruct((NP, 128), jnp.float32),
                   jax.ShapeDtypeStruct((NP, 1), jnp.float32),
                   jax.ShapeDtypeStruct((NP, 1), jnp.float32)],
    )(h1a, h1b, as1, ad1, na0, na1, nb0, nb1, d0, d1, b1a, b1b, W2a, W2b, a2s, a2d)


def _dense3_body(h2_ref, as2_ref, ad2_ref, n0_ref, n1_ref, d0_ref, d1_ref,
                 b2_ref, wl1_ref, bl1_ref, wl2_ref, bl2_ref, y_ref):
    e = as2_ref[...] + ad2_ref[...]
    wself = jnp.exp(jnp.maximum(e, 0.0) + 0.2 * jnp.minimum(e, 0.0))
    den = d0_ref[...] + d1_ref[...] + wself + 1e-16
    out2 = jnp.maximum(
        (n0_ref[...] + n1_ref[...] + wself * h2_ref[...]) / den + b2_ref[...], 0.0)
    dn = (((1,), (1,)), ((), ()))
    t = jnp.maximum(
        lax.dot_general(out2, wl1_ref[...], dn, preferred_element_type=jnp.float32)
        + bl1_ref[...], 0.0)
    y_ref[...] = (lax.dot_general(t, wl2_ref[...], dn, preferred_element_type=jnp.float32)
                  + bl2_ref[...])


@jax.jit
def _dense3(h2, as2, ad2, n0, n1, d0, d1, b2r, Wl1, bl1r, Wl2, bl2r):
    return pl.pallas_call(
        _dense3_body,
        grid=(NP // BLK,),
        in_specs=[_rows_spec(128), _rows_spec(1), _rows_spec(1),
                  _rows_spec(128), _rows_spec(128), _rows_spec(1), _rows_spec(1),
                  _full_spec(1, 128), _full_spec(256, 128), _full_spec(1, 256),
                  _full_spec(128, 256), _full_spec(1, 128)],
        out_specs=[_rows_spec(128)],
        out_shape=[jax.ShapeDtypeStruct((NP, 128), jnp.float32)],
    )(h2, as2, ad2, n0, n1, d0, d1, b2r, Wl1, bl1r, Wl2, bl2r)[0]


def kernel(x, edge_index, batch, W1, a_src1, a_dst1, b1,
           W2, a_src2, a_dst2, b2, Wl1, bl1, Wl2, bl2):
    del batch
    x_p = jnp.pad(x, ((0, NP - NN), (0, 0)))
    pad_e = EP - EE
    src = jnp.concatenate([edge_index[0], jnp.zeros((pad_e,), jnp.int32)])
    dst = jnp.concatenate([edge_index[1], jnp.full((pad_e,), DUMP, jnp.int32)])

    W1a, W1b = W1[:128], W1[128:]
    h1a, h1b, as1, ad1 = _dense1(
        x_p, W1a, W1b,
        a_src1[:128].reshape(128, 1), a_src1[128:].reshape(128, 1),
        a_dst1[:128].reshape(128, 1), a_dst1[128:].reshape(128, 1))

    as1v = as1.reshape(NP)
    ad1v = ad1.reshape(NP)
    numA, denA = _sc_agg(src, dst, as1v, ad1v, h1a)
    numB, _ = _sc_agg(src, dst, as1v, ad1v, h1b)

    h2, as2, ad2 = _dense2(
        h1a, h1b, as1, ad1,
        numA[0], numA[1], numB[0], numB[1],
        denA[0].reshape(NP, 1), denA[1].reshape(NP, 1),
        b1[:128].reshape(1, 128), b1[128:].reshape(1, 128),
        W2[:, :128], W2[:, 128:],
        a_src2.reshape(128, 1), a_dst2.reshape(128, 1))

    num2, den2 = _sc_agg(src, dst, as2.reshape(NP), ad2.reshape(NP), h2)

    y = _dense3(
        h2, as2, ad2, num2[0], num2[1],
        den2[0].reshape(NP, 1), den2[1].reshape(NP, 1),
        b2.reshape(1, 128), Wl1, bl1.reshape(1, 256), Wl2, bl2.reshape(1, 128))
    return y[:NN]
